# Initial kernel scaffold; baseline (speedup 1.0000x reference)
import jax
import jax.numpy as jnp
from jax.experimental import pallas as pl


def kernel(x, edge_index, edge_attr, batch, emb_table, w1_lin, b1_lin, w1_lin2, b1_lin2, w2_lin, b2_lin, w2_lin2, b2_lin2, bn1_g, bn1_b, bn2_g, bn2_b, gate_w, gate_b):
    raise NotImplementedError("write your pallas kernel here")


# speedup vs baseline: 1.5063x; 1.5063x over previous
import functools

import jax
import jax.numpy as jnp
from jax import lax
from jax.experimental import pallas as pl
from jax.experimental.pallas import tpu as pltpu
from jax.experimental.pallas import tpu_sc as plsc

N = 10000
D = 128
G = 64
NB = 10240
NPAD = 10240
C = 128
NC = 2
NS = 16
NW = NC * NS


def _leaky(t):
    return jnp.maximum(t, 0.01 * t)


def _prologue_body(x_ref, emb_ref, w1_ref, wl2a_ref, bl2a_ref, b1_ref,
                   w2_ref, wl2b_ref, bl2b_ref, b2_ref,
                   hw1_ref, wc1_ref, wc2_ref):
    x_blk = x_ref[...]
    blk = x_blk.shape[0]
    iota = lax.broadcasted_iota(jnp.int32, (blk, 64), 1)
    onehot = (iota == x_blk).astype(jnp.float32)
    h_blk = lax.dot_general(onehot, emb_ref[...],
                            (((1,), (0,)), ((), ())),
                            precision=lax.Precision.HIGHEST)
    hw1_ref[...] = lax.dot_general(h_blk, w1_ref[...],
                                   (((1,), (0,)), ((), ())),
                                   precision=lax.Precision.HIGHEST)
    wa = lax.dot_general(wl2a_ref[...], w1_ref[...], (((1,), (0,)), ((), ())),
                         precision=lax.Precision.HIGHEST)
    ca = lax.dot_general(bl2a_ref[...], w1_ref[...], (((1,), (0,)), ((), ())),
                         precision=lax.Precision.HIGHEST) + b1_ref[...]
    wc1_ref[...] = jnp.concatenate(
        [wa, ca, jnp.zeros((3, D), jnp.float32)], axis=0)
    wb = lax.dot_general(wl2b_ref[...], w2_ref[...], (((1,), (0,)), ((), ())),
                         precision=lax.Precision.HIGHEST)
    cb = lax.dot_general(bl2b_ref[...], w2_ref[...], (((1,), (0,)), ((), ())),
                         precision=lax.Precision.HIGHEST) + b2_ref[...]
    wc2_ref[...] = jnp.concatenate(
        [wb, cb, jnp.zeros((3, D), jnp.float32)], axis=0)


def _prologue(x_pad, emb_pad, w1, wl2a, bl2a, b1, w2, wl2b, bl2b, b2):
    blk = 512
    grid = NB // blk
    full = lambda shape: pl.BlockSpec(shape, lambda i: (0, 0))
    return pl.pallas_call(
        _prologue_body,
        grid=(grid,),
        in_specs=[
            pl.BlockSpec((blk, 1), lambda i: (i, 0)),
            full((64, D)), full((D, D)), full((4, D)), full((1, D)),
            full((1, D)), full((D, D)), full((4, D)), full((1, D)),
            full((1, D)),
        ],
        out_specs=[
            pl.BlockSpec((blk, D), lambda i: (i, 0)),
            full((8, D)), full((8, D)),
        ],
        out_shape=[
            jax.ShapeDtypeStruct((NB, D), jnp.float32),
            jax.ShapeDtypeStruct((8, D), jnp.float32),
            jax.ShapeDtypeStruct((8, D), jnp.float32),
        ],
    )(x_pad, emb_pad, w1, wl2a, bl2a, b1, w2, wl2b, bl2b, b2)


def _egnn_sc(hw, src_p, dst_p, ea_p, wc):
    e_pad = src_p.shape[0]
    chunks_per_worker = e_pad // (NW * C)
    mesh = plsc.VectorSubcoreMesh(core_axis_name="c", subcore_axis_name="s")

    @functools.partial(
        pl.kernel, mesh=mesh,
        out_type=jax.ShapeDtypeStruct((NC, N, D), jnp.float32),
        scratch_types=[
            pltpu.VMEM((C,), jnp.int32),
            pltpu.VMEM((C,), jnp.int32),
            pltpu.VMEM((C * 4,), jnp.float32),
            pltpu.VMEM((C, D), jnp.float32),
            pltpu.VMEM((8, D), jnp.float32),
            pltpu.VMEM_SHARED((NPAD, D), jnp.float32),
            pltpu.SemaphoreType.DMA,
        ],
    )
    def k(hw_hbm, src_hbm, dst_hbm, ea_hbm, wc_hbm, out_hbm,
          src_v, dst_v, ea_v, rows_v, wc_v, acc_sh, sem):
        cid = lax.axis_index("c")
        sid = lax.axis_index("s")
        wid = cid * NS + sid

        def zrow(r, _):
            for j in range(8):
                rows_v[r, pl.ds(j * 16, 16)] = jnp.zeros((16,), jnp.float32)
            return 0
        lax.fori_loop(0, C, zrow, 0)
        rows_per_sub = NPAD // NS
        def zchunk(t, _):
            pltpu.sync_copy(rows_v,
                            acc_sh.at[pl.ds(sid * rows_per_sub + t * C, C)])
            return 0
        lax.fori_loop(0, rows_per_sub // C, zchunk, 0)
        pltpu.sync_copy(wc_hbm, wc_v)
        plsc.subcore_barrier()

        wreg = [[wc_v[kk, pl.ds(j * 16, 16)] for j in range(8)]
                for kk in range(5)]

        base0 = wid * chunks_per_worker * C

        def chunk_body(t, _):
            base = base0 + t * C
            pltpu.sync_copy(src_hbm.at[pl.ds(base, C)], src_v)
            pltpu.sync_copy(dst_hbm.at[pl.ds(base, C)], dst_v)
            pltpu.sync_copy(ea_hbm.at[pl.ds(base * 4, C * 4)], ea_v)
            pltpu.async_copy(hw_hbm.at[src_v], rows_v, sem).wait()

            def quad_body(q, _):
                av = ea_v[pl.ds(q * 16, 16)]
                for tt in range(4):
                    e = q * 4 + tt
                    a0 = av[4 * tt + 0]
                    a1 = av[4 * tt + 1]
                    a2 = av[4 * tt + 2]
                    a3 = av[4 * tt + 3]
                    for j in range(8):
                        sl = pl.ds(j * 16, 16)
                        t0 = rows_v[e, sl] + wreg[4][j]
                        t0 = t0 + a0 * wreg[0][j]
                        t0 = t0 + a1 * wreg[1][j]
                        t0 = t0 + a2 * wreg[2][j]
                        t0 = t0 + a3 * wreg[3][j]
                        rows_v[e, sl] = jnp.maximum(t0, 0.01 * t0)
                return 0
            lax.fori_loop(0, C // 4, quad_body, 0)
            pltpu.sync_copy(rows_v, acc_sh.at[dst_v], add=True)
            return 0
        lax.fori_loop(0, chunks_per_worker, chunk_body, 0)
        plsc.subcore_barrier()

        @pl.when(sid < NS - 1)
        def _():
            start = pl.multiple_of(sid * 624, 8)
            pltpu.sync_copy(acc_sh.at[pl.ds(start, 624)],
                            out_hbm.at[cid, pl.ds(start, 624)])

        @pl.when(sid == NS - 1)
        def _():
            pltpu.sync_copy(acc_sh.at[pl.ds(9360, 640)],
                            out_hbm.at[cid, pl.ds(9360, 640)])

    return k(hw, src_p, dst_p, ea_p, wc)


def _mid_body(p_ref, w2_ref, g_ref, b_ref, out_ref):
    agg = p_ref[0] + p_ref[1]
    mu = jnp.mean(agg, axis=0, keepdims=True)
    var = jnp.mean(agg * agg, axis=0, keepdims=True) - mu * mu
    xn = (agg - mu) * lax.rsqrt(var + 1e-5) * g_ref[...] + b_ref[...]
    x1 = _leaky(xn)
    hw2 = lax.dot_general(x1, w2_ref[...], (((1,), (0,)), ((), ())),
                          precision=lax.Precision.HIGHEST)
    out_ref[pl.ds(0, N), :] = hw2
    out_ref[pl.ds(N, NB - N), :] = jnp.zeros((NB - N, D), jnp.float32)


def _mid(p, w2, g, b):
    return pl.pallas_call(
        _mid_body,
        out_shape=jax.ShapeDtypeStruct((NB, D), jnp.float32),
    )(p, w2, g, b)


def _epilogue_body(p_ref, g_ref, b_ref, gw_ref, gb_ref, batch_ref, out_ref):
    agg = p_ref[0] + p_ref[1]
    mu = jnp.mean(agg, axis=0, keepdims=True)
    var = jnp.mean(agg * agg, axis=0, keepdims=True) - mu * mu
    x2 = _leaky((agg - mu) * lax.rsqrt(var + 1e-5) * g_ref[...] + b_ref[...])
    gate = lax.dot_general(x2, gw_ref[...], (((1,), (0,)), ((), ())),
                           precision=lax.Precision.HIGHEST) + gb_ref[...]
    batch_col = batch_ref[...]
    iota = lax.broadcasted_iota(jnp.int32, (N, G), 1)
    mask = (iota == batch_col)
    maskf = mask.astype(jnp.float32)
    gmax = jnp.max(jnp.where(mask, gate, -1e30), axis=0, keepdims=True)
    gm_b = jnp.sum(maskf * gmax, axis=1, keepdims=True)
    gexp = jnp.exp(gate - gm_b)
    gsum = jnp.sum(maskf * gexp, axis=0, keepdims=True)
    denom = jnp.sum(maskf * gsum, axis=1, keepdims=True)
    alpha = gexp / denom
    out_ref[...] = lax.dot_general(maskf * alpha, x2,
                                   (((0,), (0,)), ((), ())),
                                   precision=lax.Precision.HIGHEST)


def _epilogue(p2, g, b, gw, gb, batch_col):
    return pl.pallas_call(
        _epilogue_body,
        out_shape=jax.ShapeDtypeStruct((G, D), jnp.float32),
    )(p2, g, b, gw, gb, batch_col)


def kernel(x, edge_index, edge_attr, batch, emb_table, w1_lin, b1_lin,
           w1_lin2, b1_lin2, w2_lin, b2_lin, w2_lin2, b2_lin2,
           bn1_g, bn1_b, bn2_g, bn2_b, gate_w, gate_b):
    e = edge_index.shape[1]
    chunks_per_worker = -(-e // (NW * C))
    e_pad = chunks_per_worker * NW * C

    src = edge_index[0]
    dst = edge_index[1]
    src_p = jnp.pad(src, (0, e_pad - e))
    dst_p = jnp.pad(dst, (0, e_pad - e), constant_values=N)
    ea_p = jnp.pad(edge_attr, ((0, e_pad - e), (0, 0))).reshape(e_pad * 4)

    x_pad = jnp.pad(x, ((0, NB - N), (0, 0)))
    emb_pad = jnp.pad(emb_table, ((0, 64 - emb_table.shape[0]), (0, 0)))

    hw1, wc1, wc2 = _prologue(
        x_pad, emb_pad, w1_lin, w1_lin2, b1_lin2.reshape(1, D),
        b1_lin.reshape(1, D), w2_lin, w2_lin2, b2_lin2.reshape(1, D),
        b2_lin.reshape(1, D))

    p1 = _egnn_sc(hw1, src_p, dst_p, ea_p, wc1)
    hw2 = _mid(p1, w2_lin, bn1_g.reshape(1, D), bn1_b.reshape(1, D))
    p2 = _egnn_sc(hw2, src_p, dst_p, ea_p, wc2)
    out = _epilogue(p2, bn2_g.reshape(1, D), bn2_b.reshape(1, D),
                    gate_w, gate_b.reshape(1, 1), batch.reshape(N, 1))
    return out
